# Initial kernel scaffold; baseline (speedup 1.0000x reference)
import jax
import jax.numpy as jnp
from jax.experimental import pallas as pl


def kernel(x, params, edge_index, batch):
    raise NotImplementedError("write your pallas kernel here")


# speedup vs baseline: 9.4780x; 9.4780x over previous
import functools

import jax
import jax.numpy as jnp
from jax import lax
from jax.experimental import pallas as pl
from jax.experimental.pallas import tpu as pltpu
from jax.experimental.pallas import tpu_sc as plsc

NUM_GRAPHS = 200
_K = 128
_NW = 32
_BLK = 1000


def _cdiv(a, b):
    return (a + b - 1) // b


@functools.lru_cache(maxsize=None)
def _make_scatter(h, nchunk, acc_rows):
    mesh = plsc.VectorSubcoreMesh(core_axis_name="c", subcore_axis_name="s")
    rows_per = acc_rows // 16

    @functools.partial(
        pl.kernel,
        out_type=jax.ShapeDtypeStruct((2, acc_rows, h), jnp.float32),
        mesh=mesh,
        scratch_types=[
            pltpu.VMEM((nchunk, _K), jnp.int32),
            pltpu.VMEM((nchunk, _K), jnp.int32),
            pltpu.VMEM((_K, h), jnp.float32),
            pltpu.VMEM_SHARED((acc_rows, h), jnp.float32),
            pltpu.SemaphoreType.DMA,
        ],
        compiler_params=pltpu.CompilerParams(use_tc_tiling_on_sc=False),
    )
    def scatter_k(src_hbm, dst_hbm, g_hbm, zeros_hbm, out_hbm,
                  src_v, dst_v, rows_v, acc_sh, sem):
        c = lax.axis_index("c")
        s = lax.axis_index("s")
        wid = s * 2 + c
        pltpu.sync_copy(zeros_hbm.at[pl.ds(s * rows_per, rows_per)],
                        acc_sh.at[pl.ds(s * rows_per, rows_per)])
        pltpu.sync_copy(src_hbm.at[wid], src_v)
        pltpu.sync_copy(dst_hbm.at[wid], dst_v)
        plsc.subcore_barrier()

        def body(j, carry):
            pltpu.async_copy(g_hbm.at[src_v.at[j]], rows_v, sem).wait()
            pltpu.sync_copy(rows_v, acc_sh.at[dst_v.at[j]], add=True)
            return carry

        lax.fori_loop(0, nchunk, body, 0)
        plsc.subcore_barrier()
        pltpu.sync_copy(acc_sh.at[pl.ds(s * rows_per, rows_per)],
                        out_hbm.at[c, pl.ds(s * rows_per, rows_per)])

    return scatter_k


def _scatter_partials(src_p, dst_p, g, zeros, nchunk, acc_rows):
    h = g.shape[1]
    return _make_scatter(h, nchunk, acc_rows)(src_p, dst_p, g, zeros)


def _mm0_body(x_ref, w_ref, o_ref):
    o_ref[...] = jnp.dot(x_ref[...], w_ref[...],
                         preferred_element_type=jnp.float32)


def _mm0(x, w):
    n, d = x.shape
    h = w.shape[1]
    grid = n // _BLK
    return pl.pallas_call(
        _mm0_body,
        grid=(grid,),
        in_specs=[pl.BlockSpec((_BLK, d), lambda i: (i, 0)),
                  pl.BlockSpec((d, h), lambda i: (0, 0))],
        out_specs=pl.BlockSpec((_BLK, h), lambda i: (i, 0)),
        out_shape=jax.ShapeDtypeStruct((n, h), jnp.float32),
    )(x, w)


def _tail(g_ref, s_ref, b1_ref, sc_ref, sh_ref, w2_ref, b2_ref):
    t = g_ref[...] + s_ref[0] + s_ref[1] + b1_ref[...]
    t = jnp.maximum(t * sc_ref[...] + sh_ref[...], 0.0)
    return jnp.maximum(
        jnp.dot(t, w2_ref[...], preferred_element_type=jnp.float32)
        + b2_ref[...], 0.0)


def _mid_body(g_ref, s_ref, b1_ref, sc_ref, sh_ref, w2_ref, b2_ref,
              w1n_ref, o_ref):
    hh = _tail(g_ref, s_ref, b1_ref, sc_ref, sh_ref, w2_ref, b2_ref)
    o_ref[...] = jnp.dot(hh, w1n_ref[...], preferred_element_type=jnp.float32)


def _mid(g, s, b1, sc, sh, w2, b2, w1n):
    n, h = g.shape
    acc_rows = s.shape[1]
    grid = n // _BLK
    vec = pl.BlockSpec((1, h), lambda i: (0, 0))
    mat = pl.BlockSpec((h, h), lambda i: (0, 0))
    return pl.pallas_call(
        _mid_body,
        grid=(grid,),
        in_specs=[pl.BlockSpec((_BLK, h), lambda i: (i, 0)),
                  pl.BlockSpec((2, _BLK, h), lambda i: (0, i, 0)),
                  vec, vec, vec, mat, vec, mat],
        out_specs=pl.BlockSpec((_BLK, h), lambda i: (i, 0)),
        out_shape=jax.ShapeDtypeStruct((n, h), jnp.float32),
    )(g, s, b1, sc, sh, w2, b2, w1n)


def _final_body(g_ref, s_ref, b1_ref, sc_ref, sh_ref, w2_ref, b2_ref,
                batch_ref, l1w_ref, l1b_ref, l2w_ref, l2b_ref,
                o_ref, pooled):
    i = pl.program_id(0)
    ng = pl.num_programs(0)
    hh = _tail(g_ref, s_ref, b1_ref, sc_ref, sh_ref, w2_ref, b2_ref)
    onehot_t = (batch_ref[0] ==
                lax.broadcasted_iota(jnp.int32, (NUM_GRAPHS, _BLK), 0)
                ).astype(jnp.float32)
    part = jnp.dot(onehot_t, hh, preferred_element_type=jnp.float32)

    @pl.when(i == 0)
    def _():
        pooled[...] = part

    @pl.when(i > 0)
    def _():
        pooled[...] += part

    @pl.when(i == ng - 1)
    def _():
        z = jnp.maximum(
            jnp.dot(pooled[...], l1w_ref[...],
                    preferred_element_type=jnp.float32) + l1b_ref[...], 0.0)
        z = jnp.dot(z, l2w_ref[...],
                    preferred_element_type=jnp.float32) + l2b_ref[...]
        m = jnp.max(z, axis=-1, keepdims=True)
        lse = m + jnp.log(jnp.sum(jnp.exp(z - m), axis=-1, keepdims=True))
        o_ref[...] = z - lse


def _final(g, s, b1, sc, sh, w2, b2, batch_row, l1w, l1b, l2w, l2b):
    n, h = g.shape
    ncls = l2w.shape[1]
    grid = n // _BLK
    vec = pl.BlockSpec((1, h), lambda i: (0, 0))
    mat = pl.BlockSpec((h, h), lambda i: (0, 0))
    return pl.pallas_call(
        _final_body,
        grid=(grid,),
        in_specs=[pl.BlockSpec((_BLK, h), lambda i: (i, 0)),
                  pl.BlockSpec((2, _BLK, h), lambda i: (0, i, 0)),
                  vec, vec, vec, mat, vec,
                  pl.BlockSpec((1, 1, _BLK), lambda i: (i, 0, 0)),
                  mat, vec,
                  pl.BlockSpec((h, ncls), lambda i: (0, 0)),
                  pl.BlockSpec((1, ncls), lambda i: (0, 0))],
        out_specs=pl.BlockSpec((NUM_GRAPHS, ncls), lambda i: (0, 0)),
        out_shape=jax.ShapeDtypeStruct((NUM_GRAPHS, ncls), jnp.float32),
        scratch_shapes=[pltpu.VMEM((NUM_GRAPHS, h), jnp.float32)],
    )(g, s, b1, sc, sh, w2, b2, batch_row, l1w, l1b, l2w, l2b)


def kernel(x, params, edge_index, batch):
    n, _ = x.shape
    convs = params['convs']
    h = convs[0]['W1'].shape[1]
    e = edge_index.shape[1]
    epw = _cdiv(e, _NW * _K) * _K
    nchunk = epw // _K
    cap = epw * _NW
    acc_rows = _cdiv(n + 1, 128) * 128

    src = edge_index[0].astype(jnp.int32)
    dst = edge_index[1].astype(jnp.int32)
    pad = cap - e
    src_p = jnp.concatenate(
        [src, jnp.zeros((pad,), jnp.int32)]).reshape(_NW, nchunk, _K)
    dst_p = jnp.concatenate(
        [dst, jnp.full((pad,), n, jnp.int32)]).reshape(_NW, nchunk, _K)
    zeros = jnp.zeros((acc_rows, h), jnp.float32)
    batch_row = batch.astype(jnp.int32).reshape(n // _BLK, 1, _BLK)

    g = _mm0(x, convs[0]['W1'])
    out = None
    for i, p in enumerate(convs):
        s = _scatter_partials(src_p, dst_p, g, zeros, nchunk, acc_rows)
        b1 = p['b1'].reshape(1, h)
        scale = (p['bn_gamma'] / jnp.sqrt(p['bn_var'] + 1e-5)).reshape(1, h)
        shift = p['bn_beta'].reshape(1, h) - p['bn_mean'].reshape(1, h) * scale
        b2 = p['b2'].reshape(1, h)
        if i + 1 < len(convs):
            g = _mid(g, s, b1, scale, shift, p['W2'], b2, convs[i + 1]['W1'])
        else:
            out = _final(g, s, b1, scale, shift, p['W2'], b2, batch_row,
                         params['lin1W'], params['lin1b'].reshape(1, h),
                         params['lin2W'],
                         params['lin2b'].reshape(1, params['lin2W'].shape[1]))
    return out


# speedup vs baseline: 10.4551x; 1.1031x over previous
import functools

import jax
import jax.numpy as jnp
from jax import lax
from jax.experimental import pallas as pl
from jax.experimental.pallas import tpu as pltpu
from jax.experimental.pallas import tpu_sc as plsc

NUM_GRAPHS = 200
_K = 128
_NW = 32
_BLK = 1000


def _cdiv(a, b):
    return (a + b - 1) // b


_NBUF = 8
_LOOK = _NBUF - 2


@functools.lru_cache(maxsize=None)
def _make_scatter(h, nc0, nc1, acc_rows):
    mesh = plsc.VectorSubcoreMesh(core_axis_name="c", subcore_axis_name="s")
    rows_per = acc_rows // 16
    assert nc0 % _NBUF == 0 and nc1 % _NBUF == 0
    assert nc0 >= _NBUF and nc1 >= _NBUF

    @functools.partial(
        pl.kernel,
        out_type=jax.ShapeDtypeStruct((2, acc_rows, h), jnp.float32),
        mesh=mesh,
        scratch_types=(
            [pltpu.VMEM((nc0, _K), jnp.int32),
             pltpu.VMEM((nc0, _K), jnp.int32)]
            + [pltpu.VMEM((_K, h), jnp.float32)] * _NBUF
            + [pltpu.VMEM_SHARED((acc_rows, h), jnp.float32)]
            + [pltpu.SemaphoreType.DMA] * (2 * _NBUF + 3)
        ),
        compiler_params=pltpu.CompilerParams(use_tc_tiling_on_sc=False),
    )
    def scatter_k(src0_hbm, dst0_hbm, src1_hbm, dst1_hbm, g_hbm, zeros_hbm,
                  out_hbm, src_v, dst_v, *bufs_and_sems):
        rows_v = bufs_and_sems[:_NBUF]
        acc_sh = bufs_and_sems[_NBUF]
        gsem = bufs_and_sems[_NBUF + 1:2 * _NBUF + 1]
        ssem = bufs_and_sems[2 * _NBUF + 1:3 * _NBUF + 1]
        isem = bufs_and_sems[3 * _NBUF + 1:]
        c = lax.axis_index("c")
        s = lax.axis_index("s")
        nchunk = jnp.where(c == 0, nc0, nc1)
        zcp = pltpu.async_copy(zeros_hbm.at[pl.ds(s * rows_per, rows_per)],
                               acc_sh.at[pl.ds(s * rows_per, rows_per)],
                               isem[0])

        @pl.when(c == 0)
        def _():
            pltpu.async_copy(src0_hbm.at[s], src_v.at[pl.ds(0, nc0)],
                             isem[1]).wait()
            pltpu.async_copy(dst0_hbm.at[s], dst_v.at[pl.ds(0, nc0)],
                             isem[2]).wait()

        @pl.when(c == 1)
        def _():
            pltpu.async_copy(src1_hbm.at[s], src_v.at[pl.ds(0, nc1)],
                             isem[1]).wait()
            pltpu.async_copy(dst1_hbm.at[s], dst_v.at[pl.ds(0, nc1)],
                             isem[2]).wait()

        zcp.wait()
        plsc.subcore_barrier()

        def gather(ch, b):
            pltpu.async_copy(g_hbm.at[src_v.at[ch]], rows_v[b], gsem[b])

        def gather_wait(ch, b):
            pltpu.make_async_copy(g_hbm.at[src_v.at[ch]], rows_v[b],
                                  gsem[b]).wait()

        def scat(ch, b):
            pltpu.async_copy(rows_v[b], acc_sh.at[dst_v.at[ch]],
                             ssem[b], add=True)

        def scat_wait(ch, b):
            pltpu.make_async_copy(rows_v[b], acc_sh.at[dst_v.at[ch]],
                                  ssem[b]).wait()

        for b0 in range(_LOOK):
            gather(b0, b0)

        def body(i, carry):
            c0 = i * _NBUF
            for bb in range(_NBUF):
                ch = c0 + bb
                bg = (bb + _LOOK) % _NBUF

                @pl.when(ch >= 2)
                def _():
                    scat_wait(ch - 2, bg)

                @pl.when(ch + _LOOK < nchunk)
                def _():
                    gather(ch + _LOOK, bg)

                gather_wait(ch, bb)
                scat(ch, bb)
            return carry

        lax.fori_loop(0, nchunk // _NBUF, body, 0)
        scat_wait(nchunk - 2, _NBUF - 2)
        scat_wait(nchunk - 1, _NBUF - 1)
        plsc.subcore_barrier()
        pltpu.sync_copy(acc_sh.at[pl.ds(s * rows_per, rows_per)],
                        out_hbm.at[c, pl.ds(s * rows_per, rows_per)])

    return scatter_k


def _scatter_partials(src0, dst0, src1, dst1, g, zeros, acc_rows):
    h = g.shape[1]
    return _make_scatter(h, src0.shape[1], src1.shape[1], acc_rows)(
        src0, dst0, src1, dst1, g, zeros)


def _mm0_body(x_ref, w_ref, o_ref):
    o_ref[...] = jnp.dot(x_ref[...], w_ref[...],
                         preferred_element_type=jnp.float32)


def _mm0(x, w):
    n, d = x.shape
    h = w.shape[1]
    return pl.pallas_call(
        _mm0_body,
        grid=(n // _BLK,),
        in_specs=[pl.BlockSpec((_BLK, d), lambda i: (i, 0)),
                  pl.BlockSpec((d, h), lambda i: (0, 0))],
        out_specs=pl.BlockSpec((_BLK, h), lambda i: (i, 0)),
        out_shape=jax.ShapeDtypeStruct((n, h), jnp.float32),
    )(x, w)


def _tail(pre, h_ref, s_ref, w1_ref, b1_ref, mean_ref, var_ref, gam_ref,
          bet_ref, w2_ref, b2_ref):
    agg = s_ref[0] + s_ref[1]
    if pre:
        u = h_ref[...] + agg + b1_ref[...]
    else:
        t = h_ref[...] + agg
        u = jnp.dot(t, w1_ref[...],
                    preferred_element_type=jnp.float32) + b1_ref[...]
    u = (u - mean_ref[...]) / jnp.sqrt(var_ref[...] + 1e-5) * gam_ref[...] \
        + bet_ref[...]
    u = jnp.maximum(u, 0.0)
    return jnp.maximum(
        jnp.dot(u, w2_ref[...], preferred_element_type=jnp.float32)
        + b2_ref[...], 0.0)


def _layer_specs(d, sw, h):
    vec = pl.BlockSpec((1, h), lambda i: (0, 0))
    return [pl.BlockSpec((_BLK, d), lambda i: (i, 0)),
            pl.BlockSpec((2, _BLK, sw), lambda i: (0, i, 0)),
            pl.BlockSpec((d, h), lambda i: (0, 0)),
            vec, vec, vec, vec, vec,
            pl.BlockSpec((h, h), lambda i: (0, 0)), vec]


def _mid(pre, hx, s, w1, b1, mean, var, gam, bet, w2, b2):
    n, d = hx.shape
    sw = s.shape[2]
    h = w1.shape[1]
    grid = n // _BLK
    return pl.pallas_call(
        functools.partial(_tail_to_out, pre),
        grid=(grid,),
        in_specs=_layer_specs(d, sw, h),
        out_specs=pl.BlockSpec((_BLK, h), lambda i: (i, 0)),
        out_shape=jax.ShapeDtypeStruct((n, h), jnp.float32),
    )(hx, s, w1, b1, mean, var, gam, bet, w2, b2)


def _tail_to_out(pre, h_ref, s_ref, w1_ref, b1_ref, mean_ref, var_ref,
                 gam_ref, bet_ref, w2_ref, b2_ref, o_ref):
    o_ref[...] = _tail(pre, h_ref, s_ref, w1_ref, b1_ref, mean_ref,
                       var_ref, gam_ref, bet_ref, w2_ref, b2_ref)


def _final_body(pre, h_ref, s_ref, w1_ref, b1_ref, mean_ref, var_ref,
                gam_ref, bet_ref, w2_ref, b2_ref,
                batch_ref, l1w_ref, l1b_ref, l2w_ref, l2b_ref,
                o_ref, pooled):
    i = pl.program_id(0)
    ng = pl.num_programs(0)
    hh = _tail(pre, h_ref, s_ref, w1_ref, b1_ref, mean_ref, var_ref,
               gam_ref, bet_ref, w2_ref, b2_ref)
    onehot_t = (batch_ref[0] ==
                lax.broadcasted_iota(jnp.int32, (NUM_GRAPHS, _BLK), 0)
                ).astype(jnp.float32)
    part = jnp.dot(onehot_t, hh, preferred_element_type=jnp.float32)

    @pl.when(i == 0)
    def _():
        pooled[...] = part

    @pl.when(i > 0)
    def _():
        pooled[...] += part

    @pl.when(i == ng - 1)
    def _():
        z = jnp.maximum(
            jnp.dot(pooled[...], l1w_ref[...],
                    preferred_element_type=jnp.float32) + l1b_ref[...], 0.0)
        z = jnp.dot(z, l2w_ref[...],
                    preferred_element_type=jnp.float32) + l2b_ref[...]
        m = jnp.max(z, axis=-1, keepdims=True)
        lse = m + jnp.log(jnp.sum(jnp.exp(z - m), axis=-1, keepdims=True))
        o_ref[...] = z - lse


def _final(pre, hx, s, w1, b1, mean, var, gam, bet, w2, b2,
           batch_row, l1w, l1b, l2w, l2b):
    n, d = hx.shape
    sw = s.shape[2]
    h = w1.shape[1]
    ncls = l2w.shape[1]
    grid = n // _BLK
    vec = pl.BlockSpec((1, h), lambda i: (0, 0))
    return pl.pallas_call(
        functools.partial(_final_body, pre),
        grid=(grid,),
        in_specs=_layer_specs(d, sw, h) + [
            pl.BlockSpec((1, 1, _BLK), lambda i: (i, 0, 0)),
            pl.BlockSpec((h, h), lambda i: (0, 0)), vec,
            pl.BlockSpec((h, ncls), lambda i: (0, 0)),
            pl.BlockSpec((1, ncls), lambda i: (0, 0))],
        out_specs=pl.BlockSpec((NUM_GRAPHS, ncls), lambda i: (0, 0)),
        out_shape=jax.ShapeDtypeStruct((NUM_GRAPHS, ncls), jnp.float32),
        scratch_shapes=[pltpu.VMEM((NUM_GRAPHS, h), jnp.float32)],
    )(hx, s, w1, b1, mean, var, gam, bet, w2, b2,
      batch_row, l1w, l1b, l2w, l2b)


def _chunked_edges(v, workers, nchunk, fill):
    cap = workers * nchunk * _K
    pad = jnp.full((cap - v.shape[0],), fill, jnp.int32)
    return jnp.concatenate([v, pad]).reshape(workers, nchunk, _K)


def kernel(x, params, edge_index, batch):
    n, d_in = x.shape
    convs = params['convs']
    h = convs[0]['W1'].shape[1]
    e = edge_index.shape[1]
    acc_rows = _cdiv(n + 1, 128) * 128

    src = edge_index[0].astype(jnp.int32)
    dst = edge_index[1].astype(jnp.int32)
    chunks_needed = _cdiv(e, 16 * _K)
    nc1 = _cdiv(_cdiv(chunks_needed, 5), _NBUF) * _NBUF
    nc0 = _cdiv(max(chunks_needed - nc1, 1), _NBUF) * _NBUF
    e0 = min(16 * nc0 * _K, e)
    src0 = _chunked_edges(src[:e0], 16, nc0, 0)
    dst0 = _chunked_edges(dst[:e0], 16, nc0, n)
    src1 = _chunked_edges(src[e0:], 16, nc1, 0)
    dst1 = _chunked_edges(dst[e0:], 16, nc1, n)
    zeros = jnp.zeros((acc_rows, h), jnp.float32)
    batch_row = batch.astype(jnp.int32).reshape(n // _BLK, 1, _BLK)

    hx = _mm0(x, convs[0]['W1'])
    out = None
    for i, p in enumerate(convs):
        s = _scatter_partials(src0, dst0, src1, dst1, hx, zeros, acc_rows)
        pre = (i == 0)
        args = (hx, s, p['W1'], p['b1'].reshape(1, h),
                p['bn_mean'].reshape(1, h), p['bn_var'].reshape(1, h),
                p['bn_gamma'].reshape(1, h), p['bn_beta'].reshape(1, h),
                p['W2'], p['b2'].reshape(1, h))
        if i + 1 < len(convs):
            hx = _mid(pre, *args)
        else:
            out = _final(pre, *args, batch_row,
                         params['lin1W'], params['lin1b'].reshape(1, h),
                         params['lin2W'],
                         params['lin2b'].reshape(1, params['lin2W'].shape[1]))
    return out
